# baseline (speedup 1.0000x reference)
import jax
import jax.numpy as jnp
from jax import lax
from jax.experimental import pallas as pl
from jax.experimental.pallas import tpu as pltpu
from jax.experimental.pallas import tpu_sc as plsc

_SRC = ["ind", "org", "ext", "ind", "org", "ext", "ind",
        "ind", "org", "ext", "ind", "org", "ext", "org"]
_DST = ["ind", "ind", "ind", "org", "org", "org", "org",
        "ind", "ind", "ind", "org", "org", "org", "ind"]
_NREL = 14

_SRC_COL = {}
for _t in ("ind", "org", "ext"):
    for _c, _r in enumerate([i for i in range(_NREL) if _SRC[i] == _t]):
        _SRC_COL[_r] = _c
_DST_SLOT = {}
for _t in ("ind", "org"):
    for _c, _r in enumerate([i for i in range(_NREL) if _DST[i] == _t]):
        _DST_SLOT[_r] = _c
_CORE = {r: (0 if _DST[r] == "ind" else 1) for r in range(_NREL)}

_E = 40000
_CH = 128
_NFULL = _E // _CH
_TAIL = _E - _NFULL * _CH
_NSUB = 16
_KMAX = -(-_NFULL // _NSUB)
_NP = 102400
_ZCH = 2048


def _project_kernel(a_ref, x_ref, *o_refs):
    res = lax.dot_general(
        a_ref[...], x_ref[...], (((1,), (1,)), ((), ())),
        preferred_element_type=jnp.float32)
    for j, o_ref in enumerate(o_refs):
        o_ref[...] = res[j, :]


def _project(x, at, ncols, bn=8192):
    n = x.shape[0]
    grid = -(-n // bn)
    vec = pl.BlockSpec((bn,), lambda i: (i,))
    return pl.pallas_call(
        _project_kernel,
        grid=(grid,),
        in_specs=[
            pl.BlockSpec((8, 128), lambda i: (0, 0)),
            pl.BlockSpec((bn, 128), lambda i: (i, 0)),
        ],
        out_specs=[vec] * ncols,
        out_shape=[jax.ShapeDtypeStruct((n,), jnp.float32)] * ncols,
    )(at, x)


def _combine_kernel(*refs):
    sums = refs[0:7]
    cnts = refs[7:14]
    y_ref, b_ref, o_ref = refs[14], refs[15], refs[16]
    tot = y_ref[...] + b_ref[0, 0]
    for j in range(7):
        tot = tot + sums[j][...] / jnp.maximum(cnts[j][...], 1.0)
    o_ref[...] = jax.nn.sigmoid(tot)


def _combine(sums, cnts, z, bsum, bn=8192):
    n = z.shape[0]
    grid = -(-n // bn)
    vec = pl.BlockSpec((bn,), lambda i: (i,))
    return pl.pallas_call(
        _combine_kernel,
        grid=(grid,),
        in_specs=[vec] * 15 + [pl.BlockSpec(memory_space=pltpu.SMEM)],
        out_specs=vec,
        out_shape=jax.ShapeDtypeStruct((n,), jnp.float32),
    )(*sums, *cnts, z, bsum)


def _make_segment_body(mode):
    with_y = mode == "sum"

    def body(*refs):
        k = 0
        if with_y:
            ys = refs[0:_NREL]
            k = _NREL
        eis = refs[k:k + _NREL]
        zeros_hbm = refs[k + _NREL]
        if not with_y:
            ones_hbm = refs[k + _NREL + 1]
            k += 1
        o_ind = refs[k + _NREL + 1:k + _NREL + 8]
        o_org = refs[k + _NREL + 8:k + _NREL + 15]
        sc = refs[k + _NREL + 15:]
        accs = sc[0:7]
        sc = sc[7:]
        if with_y:
            idx0_all, idx1_all, vals_all, idx0t, idx1t, valst = sc[0:6]
            sc = sc[6:]
        else:
            idx1_all, ones, idx1t, onest = sc[0:4]
            sc = sc[4:]
        wbufa, wbufb = sc[0:2]
        zbuf = wbufa.at[pl.ds(0, _ZCH)]
        es = sc[2:5]
        vs = sc[5:8]
        ws = sc[8:10]
        zsem = sc[10]

        c = lax.axis_index("c")
        s = lax.axis_index("s")

        core_rels = ([r for r in range(_NREL) if _CORE[r] == 0],
                     [r for r in range(_NREL) if _CORE[r] == 1])
        toff = _NFULL * _CH

        def for_chunks(fn):
            @pl.loop(0, _KMAX)
            def _(k):
                j = k * _NSUB + s

                @pl.when(j < _NFULL)
                def _():
                    fn(k, j)

        def edge_copies(i, p, go):
            for r in (core_rels[0][i], core_rels[1][i]):
                ei = eis[r]

                @pl.when(c == _CORE[r])
                def _():
                    def f(k, j):
                        off = j * _CH
                        if with_y:
                            go(ei.at[0, pl.ds(off, _CH)], idx0_all.at[p, k],
                               es[p])
                        go(ei.at[1, pl.ds(off, _CH)], idx1_all.at[p, k],
                           es[p])
                    for_chunks(f)

                    @pl.when(s == r)
                    def _():
                        if with_y:
                            go(ei.at[0, pl.ds(toff, _TAIL)], idx0t, es[p])
                        go(ei.at[1, pl.ds(toff, _TAIL)], idx1t, es[p])

        def gather_copies(i, p, go):
            for r in (core_rels[0][i], core_rels[1][i]):
                y = ys[r]

                @pl.when(c == _CORE[r])
                def _():
                    for_chunks(lambda k, j: go(
                        y.at[idx0_all.at[p, k]], vals_all.at[p, k], vs[p]))

                    @pl.when(s == r)
                    def _():
                        go(y.at[idx0t], valst, vs[p])

        def scatter_copies(i, p, go):
            for r in (core_rels[0][i], core_rels[1][i]):
                acc = accs[_DST_SLOT[r]]

                @pl.when(c == _CORE[r])
                def _():
                    if with_y:
                        for_chunks(lambda k, j: go(
                            vals_all.at[p, k], acc.at[idx1_all.at[p, k]],
                            vs[p]))

                        @pl.when(s == r)
                        def _():
                            go(valst, acc.at[idx1t], vs[p])
                    else:
                        for_chunks(lambda k, j: go(
                            ones, acc.at[idx1_all.at[p, k]], vs[p]))

                        @pl.when(s == r)
                        def _():
                            go(onest, acc.at[idx1t], vs[p])

        def fire(a, b, sem):
            pltpu.async_copy(a, b, sem)

        def fire_add(a, b, sem):
            pltpu.async_copy(a, b, sem, add=True)

        def drain(a, b, sem):
            pltpu.make_async_copy(a, b, sem).wait()

        edge_copies(0, 0, fire)
        if not with_y:
            pltpu.sync_copy(ones_hbm, ones)
            pltpu.sync_copy(ones_hbm.at[pl.ds(0, _TAIL)], onest)
        pltpu.sync_copy(zeros_hbm, zbuf)
        nz = _NP // _ZCH
        for phase in (fire, drain):
            for a, acc in enumerate(accs):
                @pl.loop(0, nz)
                def _(i):
                    @pl.when(((a * nz + i) % _NSUB) == s)
                    def _():
                        phase(zbuf, acc.at[pl.ds(i * _ZCH, _ZCH)], zsem)
        plsc.subcore_barrier()

        for i in range(7):
            p = i % 3
            if i + 1 < 7:
                edge_copies(i + 1, (i + 1) % 3, fire)
            edge_copies(i, p, drain)
            if with_y:
                gather_copies(i, p, fire)
                gather_copies(i, p, drain)
            scatter_copies(i, p, fire_add)
            if i >= 1:
                scatter_copies(i - 1, (i - 1) % 3, drain)
        scatter_copies(6, 6 % 3, drain)

        plsc.subcore_barrier()

        def writeout_core(core, outs, stripe, last):
            bufs = (wbufa, wbufb)

            def pieces(t, sz, off):
                return (accs[t].at[pl.ds(off, sz)],
                        bufs[t % 2].at[pl.ds(0, sz)],
                        outs[t].at[pl.ds(off, sz)], ws[t % 2])

            def both_sizes(t, fn):
                @pl.when((c == core) & (s < _NSUB - 1))
                def _():
                    fn(*pieces(t, stripe, s * stripe))

                @pl.when((c == core) & (s == _NSUB - 1))
                def _():
                    fn(*pieces(t, last, (_NSUB - 1) * stripe))

            def drain_out(a, b, o, sem):
                pltpu.make_async_copy(b, o, sem).wait()

            def move(a, b, o, sem):
                pltpu.sync_copy(a, b)
                pltpu.async_copy(b, o, sem)

            for t in range(7):
                if t >= 2:
                    both_sizes(t - 2, drain_out)
                both_sizes(t, move)
            both_sizes(5, drain_out)
            both_sizes(6, drain_out)

        writeout_core(0, o_ind, 6256, 6160)
        writeout_core(1, o_org, 3128, 3080)

    return body


def _segment_call(mode, ys, eis, zeros_hbm, ones_hbm):
    with_y = mode == "sum"
    mesh = plsc.VectorSubcoreMesh(core_axis_name="c", subcore_axis_name="s",
                                  num_cores=2, num_subcores=_NSUB)
    if with_y:
        bufs = [pltpu.VMEM((3, _KMAX, _CH), jnp.int32),
                pltpu.VMEM((3, _KMAX, _CH), jnp.int32),
                pltpu.VMEM((3, _KMAX, _CH), jnp.float32),
                pltpu.VMEM((_TAIL,), jnp.int32),
                pltpu.VMEM((_TAIL,), jnp.int32),
                pltpu.VMEM((_TAIL,), jnp.float32)]
    else:
        bufs = [pltpu.VMEM((3, _KMAX, _CH), jnp.int32),
                pltpu.VMEM((_CH,), jnp.float32),
                pltpu.VMEM((_TAIL,), jnp.int32),
                pltpu.VMEM((_TAIL,), jnp.float32)]
    f = pl.kernel(
        _make_segment_body(mode),
        out_type=(
            [jax.ShapeDtypeStruct((100000,), jnp.float32)] * 7
            + [jax.ShapeDtypeStruct((50000,), jnp.float32)] * 7
        ),
        mesh=mesh,
        scratch_types=(
            [pltpu.VMEM_SHARED((_NP,), jnp.float32) for _ in range(7)]
            + bufs
            + [pltpu.VMEM((6256,), jnp.float32),
               pltpu.VMEM((6256,), jnp.float32)]
            + [pltpu.SemaphoreType.DMA] * 11
        ),
        name=f"segment_{mode}",
    )
    args = list(ys) + list(eis) + [zeros_hbm] if with_y else \
        list(eis) + [zeros_hbm, ones_hbm]
    return f(*args)


def kernel(x_ind, x_org, x_ext, ei_ind_txn_ind, ei_org_txn_ind,
           ei_ext_txn_ind, ei_ind_txn_org, ei_org_txn_org, ei_ext_txn_org,
           ei_ind_role_org, ei_ind_rev_txn_ind, ei_org_rev_txn_ind,
           ei_ext_rev_txn_ind, ei_ind_rev_txn_org, ei_org_rev_txn_org,
           ei_ext_rev_txn_org, ei_org_rev_role_ind, edge_attr_dummy,
           Wl, bl, Wr):
    eis = [ei_ind_txn_ind, ei_org_txn_ind, ei_ext_txn_ind, ei_ind_txn_org,
           ei_org_txn_org, ei_ext_txn_org, ei_ind_role_org,
           ei_ind_rev_txn_ind, ei_org_rev_txn_ind, ei_ext_rev_txn_ind,
           ei_ind_rev_txn_org, ei_org_rev_txn_org, ei_ext_rev_txn_org,
           ei_org_rev_role_ind]
    x = {"ind": x_ind, "org": x_org, "ext": x_ext}

    zeros_hbm = jnp.zeros((_ZCH,), jnp.float32)
    ones_hbm = jnp.ones((_CH,), jnp.float32)

    cnts = _segment_call("cnt", None, eis, zeros_hbm, ones_hbm)

    ats = {}
    for t in ("ind", "org", "ext"):
        rows = [jnp.zeros((128,), jnp.float32)] * 8
        for r in range(_NREL):
            if _SRC[r] == t:
                rows[_SRC_COL[r]] = Wl[r, :, 0]
        if t != "ext":
            rows[5] = sum(Wr[r, :, 0] for r in range(_NREL) if _DST[r] == t)
        ats[t] = jnp.stack(rows)

    yt = {t: _project(x[t], ats[t], 4 if t == "ext" else 6)
          for t in ("ind", "org", "ext")}
    ys = [yt[_SRC[r]][_SRC_COL[r]] for r in range(_NREL)]

    sums = _segment_call("sum", ys, eis, zeros_hbm, ones_hbm)

    bsum = {t: jnp.sum(jnp.stack(
        [bl[r, 0] for r in range(_NREL) if _DST[r] == t])).reshape(1, 1)
        for t in ("ind", "org")}

    out_ind = _combine(sums[0:7], cnts[0:7], yt["ind"][5], bsum["ind"])
    out_org = _combine(sums[7:14], cnts[7:14], yt["org"][5], bsum["org"])
    return out_ind, out_org


# speedup vs baseline: 1.1209x; 1.1209x over previous
import jax
import jax.numpy as jnp
from jax import lax
from jax.experimental import pallas as pl
from jax.experimental.pallas import tpu as pltpu
from jax.experimental.pallas import tpu_sc as plsc

_SRC = ["ind", "org", "ext", "ind", "org", "ext", "ind",
        "ind", "org", "ext", "ind", "org", "ext", "org"]
_DST = ["ind", "ind", "ind", "org", "org", "org", "org",
        "ind", "ind", "ind", "org", "org", "org", "ind"]
_NREL = 14

_SRC_COL = {}
for _t in ("ind", "org", "ext"):
    for _c, _r in enumerate([i for i in range(_NREL) if _SRC[i] == _t]):
        _SRC_COL[_r] = _c
_DST_SLOT = {}
for _t in ("ind", "org"):
    for _c, _r in enumerate([i for i in range(_NREL) if _DST[i] == _t]):
        _DST_SLOT[_r] = _c
_CORE = {r: (0 if _DST[r] == "ind" else 1) for r in range(_NREL)}

_E = 40000
_CH = 128
_NFULL = _E // _CH
_TAIL = _E - _NFULL * _CH
_NSUB = 16
_KMAX = -(-_NFULL // _NSUB)
_NP = 102400
_ZCH = 2048


def _project_kernel(a_ref, x_ref, *o_refs):
    res = lax.dot_general(
        a_ref[...], x_ref[...], (((1,), (1,)), ((), ())),
        preferred_element_type=jnp.float32)
    for j, o_ref in enumerate(o_refs):
        o_ref[...] = res[j, :]


def _project(x, at, ncols, bn=8192):
    n = x.shape[0]
    grid = -(-n // bn)
    vec = pl.BlockSpec((bn,), lambda i: (i,))
    return pl.pallas_call(
        _project_kernel,
        grid=(grid,),
        in_specs=[
            pl.BlockSpec((8, 128), lambda i: (0, 0)),
            pl.BlockSpec((bn, 128), lambda i: (i, 0)),
        ],
        out_specs=[vec] * ncols,
        out_shape=[jax.ShapeDtypeStruct((n,), jnp.float32)] * ncols,
    )(at, x)


def _combine_kernel(*refs):
    sums = refs[0:7]
    cnts = refs[7:14]
    y_ref, b_ref, o_ref = refs[14], refs[15], refs[16]
    tot = y_ref[...] + b_ref[0, 0]
    for j in range(7):
        tot = tot + sums[j][...] / jnp.maximum(cnts[j][...], 1.0)
    o_ref[...] = jax.nn.sigmoid(tot)


def _combine(sums, cnts, z, bsum, bn=8192):
    n = z.shape[0]
    grid = -(-n // bn)
    vec = pl.BlockSpec((bn,), lambda i: (i,))
    return pl.pallas_call(
        _combine_kernel,
        grid=(grid,),
        in_specs=[vec] * 15 + [pl.BlockSpec(memory_space=pltpu.SMEM)],
        out_specs=vec,
        out_shape=jax.ShapeDtypeStruct((n,), jnp.float32),
    )(*sums, *cnts, z, bsum)


def _make_segment_body():
    with_y = True

    def body(*refs):
        ys = refs[0:_NREL]
        eis = refs[_NREL:2 * _NREL]
        zeros_hbm = refs[28]
        ones_hbm = refs[29]
        o_sum_ind = refs[30:37]
        o_cnt_ind = refs[37:44]
        o_sum_org = refs[44:51]
        o_cnt_org = refs[51:58]
        sc = refs[58:]
        sums = sc[0:7]
        cnts = sc[7:14]
        sc = sc[14:]
        idx0_all, idx1_all, vals_all, idx0t, idx1t, valst = sc[0:6]
        ones, onest = sc[6:8]
        sc = sc[8:]
        wbufa, wbufb = sc[0:2]
        zbuf = wbufa.at[pl.ds(0, _ZCH)]
        es = sc[2:5]
        vs = sc[5:8]
        cs = sc[8:11]
        ws = sc[11:13]
        zsem = sc[13]

        c = lax.axis_index("c")
        s = lax.axis_index("s")

        core_rels = ([r for r in range(_NREL) if _CORE[r] == 0],
                     [r for r in range(_NREL) if _CORE[r] == 1])
        toff = _NFULL * _CH

        def for_chunks(fn):
            @pl.loop(0, _KMAX)
            def _(k):
                j = k * _NSUB + s

                @pl.when(j < _NFULL)
                def _():
                    fn(k, j)

        def edge_copies(i, p, go):
            for r in (core_rels[0][i], core_rels[1][i]):
                ei = eis[r]

                @pl.when(c == _CORE[r])
                def _():
                    def f(k, j):
                        off = j * _CH
                        if with_y:
                            go(ei.at[0, pl.ds(off, _CH)], idx0_all.at[p, k],
                               es[p])
                        go(ei.at[1, pl.ds(off, _CH)], idx1_all.at[p, k],
                           es[p])
                    for_chunks(f)

                    @pl.when(s == r)
                    def _():
                        if with_y:
                            go(ei.at[0, pl.ds(toff, _TAIL)], idx0t, es[p])
                        go(ei.at[1, pl.ds(toff, _TAIL)], idx1t, es[p])

        def gather_copies(i, p, go):
            for r in (core_rels[0][i], core_rels[1][i]):
                y = ys[r]

                @pl.when(c == _CORE[r])
                def _():
                    for_chunks(lambda k, j: go(
                        y.at[idx0_all.at[p, k]], vals_all.at[p, k], vs[p]))

                    @pl.when(s == r)
                    def _():
                        go(y.at[idx0t], valst, vs[p])

        def scatter_copies(i, p, go):
            for r in (core_rels[0][i], core_rels[1][i]):
                sum_r = sums[_DST_SLOT[r]]
                cnt_r = cnts[_DST_SLOT[r]]

                @pl.when(c == _CORE[r])
                def _():
                    def f(k, j):
                        go(vals_all.at[p, k], sum_r.at[idx1_all.at[p, k]],
                           vs[p])
                        go(ones, cnt_r.at[idx1_all.at[p, k]], cs[p])
                    for_chunks(f)

                    @pl.when(s == r)
                    def _():
                        go(valst, sum_r.at[idx1t], vs[p])
                        go(onest, cnt_r.at[idx1t], cs[p])

        def fire(a, b, sem):
            pltpu.async_copy(a, b, sem)

        def fire_add(a, b, sem):
            pltpu.async_copy(a, b, sem, add=True)

        def drain(a, b, sem):
            pltpu.make_async_copy(a, b, sem).wait()

        edge_copies(0, 0, fire)
        pltpu.sync_copy(ones_hbm, ones)
        pltpu.sync_copy(ones_hbm.at[pl.ds(0, _TAIL)], onest)
        pltpu.sync_copy(zeros_hbm, zbuf)
        nz = _NP // _ZCH
        for phase in (fire, drain):
            for a, acc in enumerate(sums + cnts):
                @pl.loop(0, nz)
                def _(i):
                    @pl.when(((a * nz + i) % _NSUB) == s)
                    def _():
                        phase(zbuf, acc.at[pl.ds(i * _ZCH, _ZCH)], zsem)
        plsc.subcore_barrier()

        for i in range(7):
            p = i % 3
            if i + 1 < 7:
                edge_copies(i + 1, (i + 1) % 3, fire)
            edge_copies(i, p, drain)
            gather_copies(i, p, fire)
            gather_copies(i, p, drain)
            scatter_copies(i, p, fire_add)
            if i >= 1:
                scatter_copies(i - 1, (i - 1) % 3, drain)
        scatter_copies(6, 6 % 3, drain)

        plsc.subcore_barrier()

        def writeout_core(core, o_sum, o_cnt, stripe, last):
            bufs = (wbufa, wbufb)
            seq = []
            for slot in range(7):
                seq.append((sums[slot], o_sum[slot]))
                seq.append((cnts[slot], o_cnt[slot]))

            def pieces(t, sz, off):
                acc, out = seq[t]
                return (acc.at[pl.ds(off, sz)],
                        bufs[t % 2].at[pl.ds(0, sz)],
                        out.at[pl.ds(off, sz)], ws[t % 2])

            def both_sizes(t, fn):
                @pl.when((c == core) & (s < _NSUB - 1))
                def _():
                    fn(*pieces(t, stripe, s * stripe))

                @pl.when((c == core) & (s == _NSUB - 1))
                def _():
                    fn(*pieces(t, last, (_NSUB - 1) * stripe))

            def drain_out(a, b, o, sem):
                pltpu.make_async_copy(b, o, sem).wait()

            def move(a, b, o, sem):
                pltpu.sync_copy(a, b)
                pltpu.async_copy(b, o, sem)

            for t in range(14):
                if t >= 2:
                    both_sizes(t - 2, drain_out)
                both_sizes(t, move)
            both_sizes(12, drain_out)
            both_sizes(13, drain_out)

        writeout_core(0, o_sum_ind, o_cnt_ind, 6256, 6160)
        writeout_core(1, o_sum_org, o_cnt_org, 3128, 3080)

    return body


def _segment_call(ys, eis, zeros_hbm, ones_hbm):
    mesh = plsc.VectorSubcoreMesh(core_axis_name="c", subcore_axis_name="s",
                                  num_cores=2, num_subcores=_NSUB)
    f = pl.kernel(
        _make_segment_body(),
        out_type=(
            [jax.ShapeDtypeStruct((100000,), jnp.float32)] * 14
            + [jax.ShapeDtypeStruct((50000,), jnp.float32)] * 14
        ),
        mesh=mesh,
        scratch_types=(
            [pltpu.VMEM_SHARED((_NP,), jnp.float32) for _ in range(14)]
            + [pltpu.VMEM((3, _KMAX, _CH), jnp.int32),
               pltpu.VMEM((3, _KMAX, _CH), jnp.int32),
               pltpu.VMEM((3, _KMAX, _CH), jnp.float32),
               pltpu.VMEM((_TAIL,), jnp.int32),
               pltpu.VMEM((_TAIL,), jnp.int32),
               pltpu.VMEM((_TAIL,), jnp.float32),
               pltpu.VMEM((_CH,), jnp.float32),
               pltpu.VMEM((_TAIL,), jnp.float32)]
            + [pltpu.VMEM((6256,), jnp.float32),
               pltpu.VMEM((6256,), jnp.float32)]
            + [pltpu.SemaphoreType.DMA] * 14
        ),
        name="segment_sc",
    )
    return f(*ys, *eis, zeros_hbm, ones_hbm)


def kernel(x_ind, x_org, x_ext, ei_ind_txn_ind, ei_org_txn_ind,
           ei_ext_txn_ind, ei_ind_txn_org, ei_org_txn_org, ei_ext_txn_org,
           ei_ind_role_org, ei_ind_rev_txn_ind, ei_org_rev_txn_ind,
           ei_ext_rev_txn_ind, ei_ind_rev_txn_org, ei_org_rev_txn_org,
           ei_ext_rev_txn_org, ei_org_rev_role_ind, edge_attr_dummy,
           Wl, bl, Wr):
    eis = [ei_ind_txn_ind, ei_org_txn_ind, ei_ext_txn_ind, ei_ind_txn_org,
           ei_org_txn_org, ei_ext_txn_org, ei_ind_role_org,
           ei_ind_rev_txn_ind, ei_org_rev_txn_ind, ei_ext_rev_txn_ind,
           ei_ind_rev_txn_org, ei_org_rev_txn_org, ei_ext_rev_txn_org,
           ei_org_rev_role_ind]
    x = {"ind": x_ind, "org": x_org, "ext": x_ext}

    zeros_hbm = jnp.zeros((_ZCH,), jnp.float32)
    ones_hbm = jnp.ones((_CH,), jnp.float32)

    ats = {}
    for t in ("ind", "org", "ext"):
        rows = [jnp.zeros((128,), jnp.float32)] * 8
        for r in range(_NREL):
            if _SRC[r] == t:
                rows[_SRC_COL[r]] = Wl[r, :, 0]
        if t != "ext":
            rows[5] = sum(Wr[r, :, 0] for r in range(_NREL) if _DST[r] == t)
        ats[t] = jnp.stack(rows)

    yt = {t: _project(x[t], ats[t], 4 if t == "ext" else 6)
          for t in ("ind", "org", "ext")}
    ys = [yt[_SRC[r]][_SRC_COL[r]] for r in range(_NREL)]

    outs = _segment_call(ys, eis, zeros_hbm, ones_hbm)
    sums = outs[0:7] + outs[14:21]
    cnts = outs[7:14] + outs[21:28]

    bsum = {t: jnp.sum(jnp.stack(
        [bl[r, 0] for r in range(_NREL) if _DST[r] == t])).reshape(1, 1)
        for t in ("ind", "org")}

    out_ind = _combine(sums[0:7], cnts[0:7], yt["ind"][5], bsum["ind"])
    out_org = _combine(sums[7:14], cnts[7:14], yt["org"][5], bsum["org"])
    return out_ind, out_org


# speedup vs baseline: 1.1277x; 1.0061x over previous
import jax
import jax.numpy as jnp
from jax import lax
from jax.experimental import pallas as pl
from jax.experimental.pallas import tpu as pltpu
from jax.experimental.pallas import tpu_sc as plsc

_SRC = ["ind", "org", "ext", "ind", "org", "ext", "ind",
        "ind", "org", "ext", "ind", "org", "ext", "org"]
_DST = ["ind", "ind", "ind", "org", "org", "org", "org",
        "ind", "ind", "ind", "org", "org", "org", "ind"]
_NREL = 14

_SRC_COL = {}
for _t in ("ind", "org", "ext"):
    for _c, _r in enumerate([i for i in range(_NREL) if _SRC[i] == _t]):
        _SRC_COL[_r] = _c
_DST_SLOT = {}
for _t in ("ind", "org"):
    for _c, _r in enumerate([i for i in range(_NREL) if _DST[i] == _t]):
        _DST_SLOT[_r] = _c
_CORE = {r: (0 if _DST[r] == "ind" else 1) for r in range(_NREL)}

_E = 40000
_CH = 128
_NFULL = _E // _CH
_TAIL = _E - _NFULL * _CH
_NSUB = 16
_KMAX = -(-_NFULL // _NSUB)
_NP = 102400
_ZCH = 2048


def _project_kernel(a_ref, x_ref, *o_refs):
    res = lax.dot_general(
        a_ref[...], x_ref[...], (((1,), (1,)), ((), ())),
        preferred_element_type=jnp.float32)
    for j, o_ref in enumerate(o_refs):
        o_ref[...] = res[j, :]


def _project(x, at, ncols, bn=8192):
    n = x.shape[0]
    grid = -(-n // bn)
    vec = pl.BlockSpec((bn,), lambda i: (i,))
    return pl.pallas_call(
        _project_kernel,
        grid=(grid,),
        in_specs=[
            pl.BlockSpec((8, 128), lambda i: (0, 0)),
            pl.BlockSpec((bn, 128), lambda i: (i, 0)),
        ],
        out_specs=[vec] * ncols,
        out_shape=[jax.ShapeDtypeStruct((n,), jnp.float32)] * ncols,
    )(at, x)


def _combine_kernel(*refs):
    sums = refs[0:7]
    cnts = refs[7:14]
    y_ref, b_ref, o_ref = refs[14], refs[15], refs[16]
    tot = y_ref[...] + b_ref[0, 0]
    for j in range(7):
        tot = tot + sums[j][...] / jnp.maximum(cnts[j][...], 1.0)
    o_ref[...] = jax.nn.sigmoid(tot)


def _combine(sums, cnts, z, bsum, bn=8192):
    n = z.shape[0]
    grid = -(-n // bn)
    vec = pl.BlockSpec((bn,), lambda i: (i,))
    return pl.pallas_call(
        _combine_kernel,
        grid=(grid,),
        in_specs=[vec] * 15 + [pl.BlockSpec(memory_space=pltpu.SMEM)],
        out_specs=vec,
        out_shape=jax.ShapeDtypeStruct((n,), jnp.float32),
    )(*sums, *cnts, z, bsum)


def _make_segment_body():
    with_y = True

    def body(*refs):
        ys = refs[0:_NREL]
        eis = refs[_NREL:2 * _NREL]
        zeros_hbm = refs[28]
        ones_hbm = refs[29]
        o_sum_ind = refs[30:37]
        o_cnt_ind = refs[37:44]
        o_sum_org = refs[44:51]
        o_cnt_org = refs[51:58]
        sc = refs[58:]
        sums = sc[0:7]
        cnts = sc[7:14]
        sc = sc[14:]
        idx0_all, idx1_all, vals_all, idx0t, idx1t, valst = sc[0:6]
        ones, onest = sc[6:8]
        sc = sc[8:]
        wbufa, wbufb = sc[0:2]
        zbuf = wbufa.at[pl.ds(0, _ZCH)]
        es = sc[2:5]
        vs = sc[5:8]
        cs = sc[8:11]
        ws = sc[11:13]
        zsem = sc[13]

        c = lax.axis_index("c")
        s = lax.axis_index("s")

        core_rels = ([r for r in range(_NREL) if _CORE[r] == 0],
                     [r for r in range(_NREL) if _CORE[r] == 1])
        toff = _NFULL * _CH

        def for_chunks(fn):
            @pl.loop(0, _KMAX)
            def _(k):
                j = k * _NSUB + s

                @pl.when(j < _NFULL)
                def _():
                    fn(k, j)

        def edge_copies(i, p, go):
            for r in (core_rels[0][i], core_rels[1][i]):
                ei = eis[r]

                @pl.when(c == _CORE[r])
                def _():
                    def f(k, j):
                        off = j * _CH
                        if with_y:
                            go(ei.at[0, pl.ds(off, _CH)], idx0_all.at[p, k],
                               es[p])
                        go(ei.at[1, pl.ds(off, _CH)], idx1_all.at[p, k],
                           es[p])
                    for_chunks(f)

                    @pl.when(s == r)
                    def _():
                        if with_y:
                            go(ei.at[0, pl.ds(toff, _TAIL)], idx0t, es[p])
                        go(ei.at[1, pl.ds(toff, _TAIL)], idx1t, es[p])

        def gather_copies(i, p, go):
            for r in (core_rels[0][i], core_rels[1][i]):
                y = ys[r]

                @pl.when(c == _CORE[r])
                def _():
                    for_chunks(lambda k, j: go(
                        y.at[idx0_all.at[p, k]], vals_all.at[p, k], vs[p]))

                    @pl.when(s == r)
                    def _():
                        go(y.at[idx0t], valst, vs[p])

        def scatter_copies(i, p, go):
            for r in (core_rels[0][i], core_rels[1][i]):
                sum_r = sums[_DST_SLOT[r]]
                cnt_r = cnts[_DST_SLOT[r]]

                @pl.when(c == _CORE[r])
                def _():
                    def f(k, j):
                        go(vals_all.at[p, k], sum_r.at[idx1_all.at[p, k]],
                           vs[p])
                        go(ones, cnt_r.at[idx1_all.at[p, k]], cs[p])
                    for_chunks(f)

                    @pl.when(s == r)
                    def _():
                        go(valst, sum_r.at[idx1t], vs[p])
                        go(onest, cnt_r.at[idx1t], cs[p])

        def fire(a, b, sem):
            pltpu.async_copy(a, b, sem)

        def fire_add(a, b, sem):
            pltpu.async_copy(a, b, sem, add=True)

        def drain(a, b, sem):
            pltpu.make_async_copy(a, b, sem).wait()

        edge_copies(0, 0, fire)
        pltpu.sync_copy(ones_hbm, ones)
        pltpu.sync_copy(ones_hbm.at[pl.ds(0, _TAIL)], onest)
        pltpu.sync_copy(zeros_hbm, zbuf)
        nz = _NP // _ZCH
        for phase in (fire, drain):
            for a, acc in enumerate(sums + cnts):
                @pl.loop(0, nz)
                def _(i):
                    @pl.when(((a * nz + i) % _NSUB) == s)
                    def _():
                        phase(zbuf, acc.at[pl.ds(i * _ZCH, _ZCH)], zsem)
        plsc.subcore_barrier()

        edge_copies(0, 0, drain)
        gather_copies(0, 0, fire)
        edge_copies(1, 1, fire)
        for i in range(7):
            p = i % 3
            gather_copies(i, p, drain)
            scatter_copies(i, p, fire_add)
            if i + 1 < 7:
                edge_copies(i + 1, (i + 1) % 3, drain)
                gather_copies(i + 1, (i + 1) % 3, fire)
            if i >= 1:
                scatter_copies(i - 1, (i - 1) % 3, drain)
            if i + 2 < 7:
                edge_copies(i + 2, (i + 2) % 3, fire)
        scatter_copies(6, 6 % 3, drain)

        plsc.subcore_barrier()

        def writeout_core(core, o_sum, o_cnt, stripe, last):
            bufs = (wbufa, wbufb)
            seq = []
            for slot in range(7):
                seq.append((sums[slot], o_sum[slot]))
                seq.append((cnts[slot], o_cnt[slot]))

            def pieces(t, sz, off):
                acc, out = seq[t]
                return (acc.at[pl.ds(off, sz)],
                        bufs[t % 2].at[pl.ds(0, sz)],
                        out.at[pl.ds(off, sz)], ws[t % 2])

            def both_sizes(t, fn):
                @pl.when((c == core) & (s < _NSUB - 1))
                def _():
                    fn(*pieces(t, stripe, s * stripe))

                @pl.when((c == core) & (s == _NSUB - 1))
                def _():
                    fn(*pieces(t, last, (_NSUB - 1) * stripe))

            def drain_out(a, b, o, sem):
                pltpu.make_async_copy(b, o, sem).wait()

            def move(a, b, o, sem):
                pltpu.sync_copy(a, b)
                pltpu.async_copy(b, o, sem)

            for t in range(14):
                if t >= 2:
                    both_sizes(t - 2, drain_out)
                both_sizes(t, move)
            both_sizes(12, drain_out)
            both_sizes(13, drain_out)

        writeout_core(0, o_sum_ind, o_cnt_ind, 6256, 6160)
        writeout_core(1, o_sum_org, o_cnt_org, 3128, 3080)

    return body


def _segment_call(ys, eis, zeros_hbm, ones_hbm):
    mesh = plsc.VectorSubcoreMesh(core_axis_name="c", subcore_axis_name="s",
                                  num_cores=2, num_subcores=_NSUB)
    f = pl.kernel(
        _make_segment_body(),
        out_type=(
            [jax.ShapeDtypeStruct((100000,), jnp.float32)] * 14
            + [jax.ShapeDtypeStruct((50000,), jnp.float32)] * 14
        ),
        mesh=mesh,
        scratch_types=(
            [pltpu.VMEM_SHARED((_NP,), jnp.float32) for _ in range(14)]
            + [pltpu.VMEM((3, _KMAX, _CH), jnp.int32),
               pltpu.VMEM((3, _KMAX, _CH), jnp.int32),
               pltpu.VMEM((3, _KMAX, _CH), jnp.float32),
               pltpu.VMEM((_TAIL,), jnp.int32),
               pltpu.VMEM((_TAIL,), jnp.int32),
               pltpu.VMEM((_TAIL,), jnp.float32),
               pltpu.VMEM((_CH,), jnp.float32),
               pltpu.VMEM((_TAIL,), jnp.float32)]
            + [pltpu.VMEM((6256,), jnp.float32),
               pltpu.VMEM((6256,), jnp.float32)]
            + [pltpu.SemaphoreType.DMA] * 14
        ),
        name="segment_sc",
    )
    return f(*ys, *eis, zeros_hbm, ones_hbm)


def kernel(x_ind, x_org, x_ext, ei_ind_txn_ind, ei_org_txn_ind,
           ei_ext_txn_ind, ei_ind_txn_org, ei_org_txn_org, ei_ext_txn_org,
           ei_ind_role_org, ei_ind_rev_txn_ind, ei_org_rev_txn_ind,
           ei_ext_rev_txn_ind, ei_ind_rev_txn_org, ei_org_rev_txn_org,
           ei_ext_rev_txn_org, ei_org_rev_role_ind, edge_attr_dummy,
           Wl, bl, Wr):
    eis = [ei_ind_txn_ind, ei_org_txn_ind, ei_ext_txn_ind, ei_ind_txn_org,
           ei_org_txn_org, ei_ext_txn_org, ei_ind_role_org,
           ei_ind_rev_txn_ind, ei_org_rev_txn_ind, ei_ext_rev_txn_ind,
           ei_ind_rev_txn_org, ei_org_rev_txn_org, ei_ext_rev_txn_org,
           ei_org_rev_role_ind]
    x = {"ind": x_ind, "org": x_org, "ext": x_ext}

    zeros_hbm = jnp.zeros((_ZCH,), jnp.float32)
    ones_hbm = jnp.ones((_CH,), jnp.float32)

    ats = {}
    for t in ("ind", "org", "ext"):
        rows = [jnp.zeros((128,), jnp.float32)] * 8
        for r in range(_NREL):
            if _SRC[r] == t:
                rows[_SRC_COL[r]] = Wl[r, :, 0]
        if t != "ext":
            rows[5] = sum(Wr[r, :, 0] for r in range(_NREL) if _DST[r] == t)
        ats[t] = jnp.stack(rows)

    yt = {t: _project(x[t], ats[t], 4 if t == "ext" else 6)
          for t in ("ind", "org", "ext")}
    ys = [yt[_SRC[r]][_SRC_COL[r]] for r in range(_NREL)]

    outs = _segment_call(ys, eis, zeros_hbm, ones_hbm)
    sums = outs[0:7] + outs[14:21]
    cnts = outs[7:14] + outs[21:28]

    bsum = {t: jnp.sum(jnp.stack(
        [bl[r, 0] for r in range(_NREL) if _DST[r] == t])).reshape(1, 1)
        for t in ("ind", "org")}

    out_ind = _combine(sums[0:7], cnts[0:7], yt["ind"][5], bsum["ind"])
    out_org = _combine(sums[7:14], cnts[7:14], yt["org"][5], bsum["org"])
    return out_ind, out_org


# speedup vs baseline: 1.1670x; 1.0348x over previous
import jax
import jax.numpy as jnp
from jax import lax
from jax.experimental import pallas as pl
from jax.experimental.pallas import tpu as pltpu
from jax.experimental.pallas import tpu_sc as plsc

_SRC = ["ind", "org", "ext", "ind", "org", "ext", "ind",
        "ind", "org", "ext", "ind", "org", "ext", "org"]
_DST = ["ind", "ind", "ind", "org", "org", "org", "org",
        "ind", "ind", "ind", "org", "org", "org", "ind"]
_NREL = 14

_SRC_COL = {}
for _t in ("ind", "org", "ext"):
    for _c, _r in enumerate([i for i in range(_NREL) if _SRC[i] == _t]):
        _SRC_COL[_r] = _c
_DST_SLOT = {}
for _t in ("ind", "org"):
    for _c, _r in enumerate([i for i in range(_NREL) if _DST[i] == _t]):
        _DST_SLOT[_r] = _c
_CORE = {r: (0 if _DST[r] == "ind" else 1) for r in range(_NREL)}

_E = 40000
_CH = 128
_NFULL = _E // _CH
_TAIL = _E - _NFULL * _CH
_NSUB = 16
_KMAX = -(-_NFULL // _NSUB)
_NP = 102400
_ZCH = 2048


def _project_kernel(a_ref, x_ref, *o_refs):
    res = lax.dot_general(
        a_ref[...], x_ref[...], (((1,), (1,)), ((), ())),
        preferred_element_type=jnp.float32)
    for j, o_ref in enumerate(o_refs):
        o_ref[...] = res[j, :]


def _project(x, at, ncols, bn=8192):
    n = x.shape[0]
    grid = -(-n // bn)
    vec = pl.BlockSpec((bn,), lambda i: (i,))
    return pl.pallas_call(
        _project_kernel,
        grid=(grid,),
        in_specs=[
            pl.BlockSpec((8, 128), lambda i: (0, 0)),
            pl.BlockSpec((bn, 128), lambda i: (i, 0)),
        ],
        out_specs=[vec] * ncols,
        out_shape=[jax.ShapeDtypeStruct((n,), jnp.float32)] * ncols,
    )(at, x)


def _combine_kernel(*refs):
    sums = refs[0:7]
    cnts = refs[7:14]
    y_ref, b_ref, o_ref = refs[14], refs[15], refs[16]
    tot = y_ref[...] + b_ref[0, 0]
    for j in range(7):
        tot = tot + sums[j][...] / jnp.maximum(cnts[j][...], 1.0)
    o_ref[...] = jax.nn.sigmoid(tot)


def _combine(sums, cnts, z, bsum, bn=16384):
    n = z.shape[0]
    grid = -(-n // bn)
    vec = pl.BlockSpec((bn,), lambda i: (i,))
    return pl.pallas_call(
        _combine_kernel,
        grid=(grid,),
        in_specs=[vec] * 15 + [pl.BlockSpec(memory_space=pltpu.SMEM)],
        out_specs=vec,
        out_shape=jax.ShapeDtypeStruct((n,), jnp.float32),
    )(*sums, *cnts, z, bsum)


def _make_segment_body():
    with_y = True

    def body(*refs):
        ys = refs[0:_NREL]
        eis = refs[_NREL:2 * _NREL]
        zeros_hbm = refs[28]
        ones_hbm = refs[29]
        o_sum_ind = refs[30:37]
        o_cnt_ind = refs[37:44]
        o_sum_org = refs[44:51]
        o_cnt_org = refs[51:58]
        sc = refs[58:]
        sums = sc[0:7]
        cnts = sc[7:14]
        sc = sc[14:]
        idx0_all, idx1_all, vals_all, idx0t, idx1t, valst = sc[0:6]
        ones, onest = sc[6:8]
        sc = sc[8:]
        wbufa, wbufb = sc[0:2]
        zbuf = wbufa.at[pl.ds(0, _ZCH)]
        es = sc[2:5]
        vs = sc[5:8]
        cs = sc[8:11]
        ws = sc[11:13]
        zsem = sc[13]

        c = lax.axis_index("c")
        s = lax.axis_index("s")

        core_rels = ([r for r in range(_NREL) if _CORE[r] == 0],
                     [r for r in range(_NREL) if _CORE[r] == 1])
        toff = _NFULL * _CH

        def for_chunks(fn):
            @pl.loop(0, _KMAX)
            def _(k):
                j = k * _NSUB + s

                @pl.when(j < _NFULL)
                def _():
                    fn(k, j)

        def edge_copies(i, p, go):
            for r in (core_rels[0][i], core_rels[1][i]):
                ei = eis[r]

                @pl.when(c == _CORE[r])
                def _():
                    def f(k, j):
                        off = j * _CH
                        if with_y:
                            go(ei.at[0, pl.ds(off, _CH)], idx0_all.at[p, k],
                               es[p])
                        go(ei.at[1, pl.ds(off, _CH)], idx1_all.at[p, k],
                           es[p])
                    for_chunks(f)

                    @pl.when(s == r)
                    def _():
                        if with_y:
                            go(ei.at[0, pl.ds(toff, _TAIL)], idx0t, es[p])
                        go(ei.at[1, pl.ds(toff, _TAIL)], idx1t, es[p])

        def gather_copies(i, p, go):
            for r in (core_rels[0][i], core_rels[1][i]):
                y = ys[r]

                @pl.when(c == _CORE[r])
                def _():
                    for_chunks(lambda k, j: go(
                        y.at[idx0_all.at[p, k]], vals_all.at[p, k], vs[p]))

                    @pl.when(s == r)
                    def _():
                        go(y.at[idx0t], valst, vs[p])

        def scatter_copies(i, p, go):
            for r in (core_rels[0][i], core_rels[1][i]):
                sum_r = sums[_DST_SLOT[r]]
                cnt_r = cnts[_DST_SLOT[r]]

                @pl.when(c == _CORE[r])
                def _():
                    def f(k, j):
                        go(vals_all.at[p, k], sum_r.at[idx1_all.at[p, k]],
                           vs[p])
                        go(ones, cnt_r.at[idx1_all.at[p, k]], cs[p])
                    for_chunks(f)

                    @pl.when(s == r)
                    def _():
                        go(valst, sum_r.at[idx1t], vs[p])
                        go(onest, cnt_r.at[idx1t], cs[p])

        def fire(a, b, sem):
            pltpu.async_copy(a, b, sem)

        def fire_add(a, b, sem):
            pltpu.async_copy(a, b, sem, add=True)

        def drain(a, b, sem):
            pltpu.make_async_copy(a, b, sem).wait()

        edge_copies(0, 0, fire)
        pltpu.sync_copy(ones_hbm, ones)
        pltpu.sync_copy(ones_hbm.at[pl.ds(0, _TAIL)], onest)
        pltpu.sync_copy(zeros_hbm, zbuf)
        nz = _NP // _ZCH
        for phase in (fire, drain):
            for a, acc in enumerate(sums + cnts):
                @pl.loop(0, nz)
                def _(i):
                    @pl.when(((a * nz + i) % _NSUB) == s)
                    def _():
                        phase(zbuf, acc.at[pl.ds(i * _ZCH, _ZCH)], zsem)
        plsc.subcore_barrier()

        edge_copies(0, 0, drain)
        gather_copies(0, 0, fire)
        edge_copies(1, 1, fire)
        for i in range(7):
            p = i % 3
            gather_copies(i, p, drain)
            scatter_copies(i, p, fire_add)
            if i + 1 < 7:
                edge_copies(i + 1, (i + 1) % 3, drain)
                gather_copies(i + 1, (i + 1) % 3, fire)
            if i >= 1:
                scatter_copies(i - 1, (i - 1) % 3, drain)
            if i + 2 < 7:
                edge_copies(i + 2, (i + 2) % 3, fire)
        scatter_copies(6, 6 % 3, drain)

        plsc.subcore_barrier()

        def writeout_core(core, o_sum, o_cnt, stripe, last):
            bufs = (wbufa, wbufb)
            seq = []
            for slot in range(7):
                seq.append((sums[slot], o_sum[slot]))
                seq.append((cnts[slot], o_cnt[slot]))

            def pieces(t, sz, off):
                acc, out = seq[t]
                return (acc.at[pl.ds(off, sz)],
                        bufs[t % 2].at[pl.ds(0, sz)],
                        out.at[pl.ds(off, sz)], ws[t % 2])

            def both_sizes(t, fn):
                @pl.when((c == core) & (s < _NSUB - 1))
                def _():
                    fn(*pieces(t, stripe, s * stripe))

                @pl.when((c == core) & (s == _NSUB - 1))
                def _():
                    fn(*pieces(t, last, (_NSUB - 1) * stripe))

            def drain_out(a, b, o, sem):
                pltpu.make_async_copy(b, o, sem).wait()

            def move(a, b, o, sem):
                pltpu.sync_copy(a, b)
                pltpu.async_copy(b, o, sem)

            for t in range(14):
                if t >= 2:
                    both_sizes(t - 2, drain_out)
                both_sizes(t, move)
            both_sizes(12, drain_out)
            both_sizes(13, drain_out)

        writeout_core(0, o_sum_ind, o_cnt_ind, 6256, 6160)
        writeout_core(1, o_sum_org, o_cnt_org, 3128, 3080)

    return body


def _segment_call(ys, eis, zeros_hbm, ones_hbm):
    mesh = plsc.VectorSubcoreMesh(core_axis_name="c", subcore_axis_name="s",
                                  num_cores=2, num_subcores=_NSUB)
    f = pl.kernel(
        _make_segment_body(),
        out_type=(
            [jax.ShapeDtypeStruct((100000,), jnp.float32)] * 14
            + [jax.ShapeDtypeStruct((50000,), jnp.float32)] * 14
        ),
        mesh=mesh,
        scratch_types=(
            [pltpu.VMEM_SHARED((_NP,), jnp.float32) for _ in range(14)]
            + [pltpu.VMEM((3, _KMAX, _CH), jnp.int32),
               pltpu.VMEM((3, _KMAX, _CH), jnp.int32),
               pltpu.VMEM((3, _KMAX, _CH), jnp.float32),
               pltpu.VMEM((_TAIL,), jnp.int32),
               pltpu.VMEM((_TAIL,), jnp.int32),
               pltpu.VMEM((_TAIL,), jnp.float32),
               pltpu.VMEM((_CH,), jnp.float32),
               pltpu.VMEM((_TAIL,), jnp.float32)]
            + [pltpu.VMEM((6256,), jnp.float32),
               pltpu.VMEM((6256,), jnp.float32)]
            + [pltpu.SemaphoreType.DMA] * 14
        ),
        name="segment_sc",
    )
    return f(*ys, *eis, zeros_hbm, ones_hbm)


def kernel(x_ind, x_org, x_ext, ei_ind_txn_ind, ei_org_txn_ind,
           ei_ext_txn_ind, ei_ind_txn_org, ei_org_txn_org, ei_ext_txn_org,
           ei_ind_role_org, ei_ind_rev_txn_ind, ei_org_rev_txn_ind,
           ei_ext_rev_txn_ind, ei_ind_rev_txn_org, ei_org_rev_txn_org,
           ei_ext_rev_txn_org, ei_org_rev_role_ind, edge_attr_dummy,
           Wl, bl, Wr):
    eis = [ei_ind_txn_ind, ei_org_txn_ind, ei_ext_txn_ind, ei_ind_txn_org,
           ei_org_txn_org, ei_ext_txn_org, ei_ind_role_org,
           ei_ind_rev_txn_ind, ei_org_rev_txn_ind, ei_ext_rev_txn_ind,
           ei_ind_rev_txn_org, ei_org_rev_txn_org, ei_ext_rev_txn_org,
           ei_org_rev_role_ind]
    x = {"ind": x_ind, "org": x_org, "ext": x_ext}

    zeros_hbm = jnp.zeros((_ZCH,), jnp.float32)
    ones_hbm = jnp.ones((_CH,), jnp.float32)

    ats = {}
    for t in ("ind", "org", "ext"):
        rows = [jnp.zeros((128,), jnp.float32)] * 8
        for r in range(_NREL):
            if _SRC[r] == t:
                rows[_SRC_COL[r]] = Wl[r, :, 0]
        if t != "ext":
            rows[5] = sum(Wr[r, :, 0] for r in range(_NREL) if _DST[r] == t)
        ats[t] = jnp.stack(rows)

    yt = {t: _project(x[t], ats[t], 4 if t == "ext" else 6)
          for t in ("ind", "org", "ext")}
    ys = [yt[_SRC[r]][_SRC_COL[r]] for r in range(_NREL)]

    outs = _segment_call(ys, eis, zeros_hbm, ones_hbm)
    sums = outs[0:7] + outs[14:21]
    cnts = outs[7:14] + outs[21:28]

    bsum = {t: jnp.sum(jnp.stack(
        [bl[r, 0] for r in range(_NREL) if _DST[r] == t])).reshape(1, 1)
        for t in ("ind", "org")}

    out_ind = _combine(sums[0:7], cnts[0:7], yt["ind"][5], bsum["ind"])
    out_org = _combine(sums[7:14], cnts[7:14], yt["org"][5], bsum["org"])
    return out_ind, out_org
